# baseline (speedup 1.0000x reference)
import jax
import jax.numpy as jnp
from jax.experimental import pallas as pl
from jax.experimental.pallas import tpu as pltpu

N_CLASSES = 10
_S = 256
_PREC = jax.lax.Precision.HIGHEST


def _feat_kernel(x_ref, t1_ref, b1_ref, t2_ref, b2_ref,
                 wf1_ref, bf1_ref, wf2_ref, bf2_ref, hwa_ref, gb_ref, wn_ref,
                 o_ref, z1_ref, z2_ref, p1_ref, p2_ref):
    S = x_ref.shape[1]

    z1_ref[...] = jnp.dot(
        x_ref[...].reshape(32 * S, 96), t1_ref[...],
        preferred_element_type=jnp.float32, precision=_PREC,
    ).reshape(32, S, 640)
    for c in range(7):
        acc = None
        for kh in range(5):
            d = z1_ref[pl.ds(4 * c + kh, 4), :, pl.ds(128 * kh, 112)]
            acc = d if acc is None else acc + d
        for q in range(2):
            u = jnp.maximum(acc[2 * q], acc[2 * q + 1])
            v = jnp.maximum(u[:, :56], u[:, 56:])
            p1_ref[2 * c + q] = jnp.maximum(v + b1_ref[...], 0.0)

    z2_ref[...] = jnp.dot(
        p1_ref[...].reshape(14 * S, 56), t2_ref[...],
        preferred_element_type=jnp.float32, precision=_PREC,
    ).reshape(14, S, 640)
    for j in range(5):
        acc = None
        for kh in range(5):
            d = z2_ref[pl.ds(2 * j + kh, 2), :, pl.ds(128 * kh, 120)]
            acc = d if acc is None else acc + d
        u = jnp.maximum(acc[0], acc[1])
        v = jnp.maximum(u[:, :60], u[:, 60:])
        p2_ref[j] = jnp.maximum(v + b2_ref[...], 0.0)

    y = bf1_ref[...]
    for h in range(5):
        y = y + jnp.dot(p2_ref[h], wf1_ref[h], preferred_element_type=jnp.float32,
                        precision=_PREC)
    y = jnp.maximum(y, 0.0)

    f = jnp.dot(y, wf2_ref[...], preferred_element_type=jnp.float32,
                precision=_PREC) + bf2_ref[...]

    z = jnp.dot(f, hwa_ref[...], preferred_element_type=jnp.float32,
                precision=_PREC)
    xn = jnp.maximum(jnp.sqrt(jnp.sum(f * f, axis=-1, keepdims=True)), 1e-8)
    hcos = z[:, :N_CLASSES] / (xn * wn_ref[...])
    gl = z[:, N_CLASSES:N_CLASSES + 1] + gb_ref[...]
    o_ref[...] = jnp.concatenate([hcos, gl], axis=1)


def _head_kernel(a_ref, o_ref):
    a = a_ref[...]
    gl = a[:, N_CLASSES:N_CLASSES + 1]
    h = a[:, :N_CLASSES]
    mu = jnp.mean(gl, axis=0, keepdims=True)
    var = jnp.mean((gl - mu) ** 2, axis=0, keepdims=True)
    g = jax.nn.sigmoid((gl - mu) * jax.lax.rsqrt(var + 1e-5))
    out = g / h
    out = out - jnp.max(out, axis=-1, keepdims=True)
    e = jnp.exp(out)
    o_ref[...] = e / jnp.sum(e, axis=-1, keepdims=True)


@jax.jit
def _forward(x, w1, b1, w2, b2, wf1, bf1, wf2, bf2, hwa, gb, wn):
    B = x.shape[0]
    S = _S
    xr = jnp.transpose(x, (2, 0, 1, 3)).reshape(32, B, 96)
    t1 = jnp.transpose(w1, (0, 2, 1, 3)).reshape(5, 32, 3, 112)
    t1 = jnp.transpose(t1, (0, 2, 1, 3)).reshape(5, 96, 112)
    t1 = jnp.pad(t1, ((0, 0), (0, 0), (0, 16)))
    t1 = jnp.transpose(t1, (1, 0, 2)).reshape(96, 640)
    t2 = jnp.transpose(w2, (0, 2, 1, 3)).reshape(5, 56, 120)
    t2 = jnp.pad(t2, ((0, 0), (0, 0), (0, 8)))
    t2 = jnp.transpose(t2, (1, 0, 2)).reshape(56, 640)

    part = pl.pallas_call(
        _feat_kernel,
        out_shape=jax.ShapeDtypeStruct((B, N_CLASSES + 1), jnp.float32),
        grid=(B // S,),
        in_specs=[
            pl.BlockSpec((32, S, 96), lambda i: (0, i, 0)),
            pl.BlockSpec((96, 640), lambda i: (0, 0)),
            pl.BlockSpec((1, 56), lambda i: (0, 0)),
            pl.BlockSpec((56, 640), lambda i: (0, 0)),
            pl.BlockSpec((1, 60), lambda i: (0, 0)),
            pl.BlockSpec((5, 60, 120), lambda i: (0, 0, 0)),
            pl.BlockSpec((1, 120), lambda i: (0, 0)),
            pl.BlockSpec((120, 64), lambda i: (0, 0)),
            pl.BlockSpec((1, 64), lambda i: (0, 0)),
            pl.BlockSpec((64, N_CLASSES + 1), lambda i: (0, 0)),
            pl.BlockSpec((1, 1), lambda i: (0, 0)),
            pl.BlockSpec((1, N_CLASSES), lambda i: (0, 0)),
        ],
        out_specs=pl.BlockSpec((S, N_CLASSES + 1), lambda i: (i, 0)),
        scratch_shapes=[pltpu.VMEM((32, S, 640), jnp.float32),
                        pltpu.VMEM((14, S, 640), jnp.float32),
                        pltpu.VMEM((14, S, 56), jnp.float32),
                        pltpu.VMEM((5, S, 60), jnp.float32)],
        compiler_params=pltpu.CompilerParams(
            dimension_semantics=("parallel",)),
    )(xr, t1, b1, t2, b2, wf1, bf1, wf2, bf2, hwa, gb, wn)

    pred = pl.pallas_call(
        _head_kernel,
        out_shape=jax.ShapeDtypeStruct((B, N_CLASSES), jnp.float32),
        grid=(1,),
        in_specs=[pl.BlockSpec((B, N_CLASSES + 1), lambda i: (0, 0))],
        out_specs=pl.BlockSpec((B, N_CLASSES), lambda i: (0, 0)),
        compiler_params=pltpu.CompilerParams(
            dimension_semantics=("arbitrary",)),
    )(part)
    return pred


def kernel(x, w1, b1, w2, b2, wf1, bf1, wf2, bf2, hwa, gb, wn):
    return _forward(x, w1, b1, w2, b2, wf1, bf1, wf2, bf2, hwa, gb, wn)


# speedup vs baseline: 1.0318x; 1.0318x over previous
import jax
import jax.numpy as jnp
from jax.experimental import pallas as pl
from jax.experimental.pallas import tpu as pltpu

N_CLASSES = 10
_S = 256
_PREC = jax.lax.Precision.HIGHEST


def _feat_kernel(x_ref, t1a_ref, t1b_ref, b1_ref, t2a_ref, t2b_ref, b2_ref,
                 wf1_ref, bf1_ref, wf2_ref, bf2_ref, hwa_ref, gb_ref, wn_ref,
                 o_ref, p1_ref, p2_ref):
    S = x_ref.shape[1]

    for c in range(7):
        lhs_a = x_ref[pl.ds(4 * c, 6)].reshape(6 * S, 96)
        za = jnp.dot(lhs_a, t1a_ref[...], preferred_element_type=jnp.float32,
                     precision=_PREC).reshape(6, S, 384)
        lhs_b = x_ref[pl.ds(4 * c + 3, 5)].reshape(5 * S, 96)
        zb = jnp.dot(lhs_b, t1b_ref[...], preferred_element_type=jnp.float32,
                     precision=_PREC).reshape(5, S, 256)
        acc = None
        for kh in range(3):
            d = za[kh:kh + 4, :, 128 * kh:128 * kh + 112]
            acc = d if acc is None else acc + d
        for kh in range(3, 5):
            acc = acc + zb[kh - 3:kh + 1, :, 128 * (kh - 3):128 * (kh - 3) + 112]
        for q in range(2):
            u = jnp.maximum(acc[2 * q], acc[2 * q + 1])
            v = jnp.maximum(u[:, :56], u[:, 56:])
            p1_ref[2 * c + q] = jnp.maximum(v + b1_ref[...], 0.0)

    for j in range(5):
        lhs_a = p1_ref[pl.ds(2 * j, 4)].reshape(4 * S, 56)
        za = jnp.dot(lhs_a, t2a_ref[...], preferred_element_type=jnp.float32,
                     precision=_PREC).reshape(4, S, 384)
        lhs_b = p1_ref[pl.ds(2 * j + 3, 3)].reshape(3 * S, 56)
        zb = jnp.dot(lhs_b, t2b_ref[...], preferred_element_type=jnp.float32,
                     precision=_PREC).reshape(3, S, 256)
        acc = None
        for kh in range(3):
            d = za[kh:kh + 2, :, 128 * kh:128 * kh + 120]
            acc = d if acc is None else acc + d
        for kh in range(3, 5):
            acc = acc + zb[kh - 3:kh - 1, :, 128 * (kh - 3):128 * (kh - 3) + 120]
        u = jnp.maximum(acc[0], acc[1])
        v = jnp.maximum(u[:, :60], u[:, 60:])
        p2_ref[j] = jnp.maximum(v + b2_ref[...], 0.0)

    y = bf1_ref[...]
    for h in range(5):
        y = y + jnp.dot(p2_ref[h], wf1_ref[h], preferred_element_type=jnp.float32,
                        precision=_PREC)
    y = jnp.maximum(y, 0.0)

    f = jnp.dot(y, wf2_ref[...], preferred_element_type=jnp.float32,
                precision=_PREC) + bf2_ref[...]

    z = jnp.dot(f, hwa_ref[...], preferred_element_type=jnp.float32,
                precision=_PREC)
    xn = jnp.maximum(jnp.sqrt(jnp.sum(f * f, axis=-1, keepdims=True)), 1e-8)
    hcos = z[:, :N_CLASSES] / (xn * wn_ref[...])
    gl = z[:, N_CLASSES:N_CLASSES + 1] + gb_ref[...]
    o_ref[...] = jnp.concatenate([hcos, gl], axis=1)


def _head_kernel(a_ref, o_ref):
    a = a_ref[...]
    gl = a[:, N_CLASSES:N_CLASSES + 1]
    h = a[:, :N_CLASSES]
    mu = jnp.mean(gl, axis=0, keepdims=True)
    var = jnp.mean((gl - mu) ** 2, axis=0, keepdims=True)
    g = jax.nn.sigmoid((gl - mu) * jax.lax.rsqrt(var + 1e-5))
    out = g / h
    out = out - jnp.max(out, axis=-1, keepdims=True)
    e = jnp.exp(out)
    o_ref[...] = e / jnp.sum(e, axis=-1, keepdims=True)


@jax.jit
def _forward(x, w1, b1, w2, b2, wf1, bf1, wf2, bf2, hwa, gb, wn):
    B = x.shape[0]
    S = _S
    xr = jnp.transpose(x, (2, 0, 1, 3)).reshape(32, B, 96)
    t1 = jnp.transpose(w1, (0, 2, 1, 3)).reshape(5, 32, 3, 112)
    t1 = jnp.transpose(t1, (0, 2, 1, 3)).reshape(5, 96, 112)
    t1 = jnp.pad(t1, ((0, 0), (0, 0), (0, 16)))
    t1a = jnp.transpose(t1[:3], (1, 0, 2)).reshape(96, 384)
    t1b = jnp.transpose(t1[3:], (1, 0, 2)).reshape(96, 256)
    t2 = jnp.transpose(w2, (0, 2, 1, 3)).reshape(5, 56, 120)
    t2 = jnp.pad(t2, ((0, 0), (0, 0), (0, 8)))
    t2a = jnp.transpose(t2[:3], (1, 0, 2)).reshape(56, 384)
    t2b = jnp.transpose(t2[3:], (1, 0, 2)).reshape(56, 256)

    part = pl.pallas_call(
        _feat_kernel,
        out_shape=jax.ShapeDtypeStruct((B, N_CLASSES + 1), jnp.float32),
        grid=(B // S,),
        in_specs=[
            pl.BlockSpec((32, S, 96), lambda i: (0, i, 0)),
            pl.BlockSpec((96, 384), lambda i: (0, 0)),
            pl.BlockSpec((96, 256), lambda i: (0, 0)),
            pl.BlockSpec((1, 56), lambda i: (0, 0)),
            pl.BlockSpec((56, 384), lambda i: (0, 0)),
            pl.BlockSpec((56, 256), lambda i: (0, 0)),
            pl.BlockSpec((1, 60), lambda i: (0, 0)),
            pl.BlockSpec((5, 60, 120), lambda i: (0, 0, 0)),
            pl.BlockSpec((1, 120), lambda i: (0, 0)),
            pl.BlockSpec((120, 64), lambda i: (0, 0)),
            pl.BlockSpec((1, 64), lambda i: (0, 0)),
            pl.BlockSpec((64, N_CLASSES + 1), lambda i: (0, 0)),
            pl.BlockSpec((1, 1), lambda i: (0, 0)),
            pl.BlockSpec((1, N_CLASSES), lambda i: (0, 0)),
        ],
        out_specs=pl.BlockSpec((S, N_CLASSES + 1), lambda i: (i, 0)),
        scratch_shapes=[pltpu.VMEM((14, S, 56), jnp.float32),
                        pltpu.VMEM((5, S, 60), jnp.float32)],
        compiler_params=pltpu.CompilerParams(
            dimension_semantics=("parallel",)),
    )(xr, t1a, t1b, b1, t2a, t2b, b2, wf1, bf1, wf2, bf2, hwa, gb, wn)

    pred = pl.pallas_call(
        _head_kernel,
        out_shape=jax.ShapeDtypeStruct((B, N_CLASSES), jnp.float32),
        grid=(1,),
        in_specs=[pl.BlockSpec((B, N_CLASSES + 1), lambda i: (0, 0))],
        out_specs=pl.BlockSpec((B, N_CLASSES), lambda i: (0, 0)),
        compiler_params=pltpu.CompilerParams(
            dimension_semantics=("arbitrary",)),
    )(part)
    return pred


def kernel(x, w1, b1, w2, b2, wf1, bf1, wf2, bf2, hwa, gb, wn):
    return _forward(x, w1, b1, w2, b2, wf1, bf1, wf2, bf2, hwa, gb, wn)


# speedup vs baseline: 1.1391x; 1.1041x over previous
import jax
import jax.numpy as jnp
from jax.experimental import pallas as pl
from jax.experimental.pallas import tpu as pltpu

N_CLASSES = 10
_S = 512
_PREC = jax.lax.Precision.HIGHEST


def _feat_kernel(x_ref, t1_ref, b1_ref, t2_ref, b2_ref,
                 wf1_ref, bf1_ref, wf2_ref, bf2_ref, hwa_ref, gb_ref, wn_ref,
                 o_ref, p1_ref, p2_ref):
    S = x_ref.shape[1]

    for c in range(7):
        acc = None
        for kh in range(5):
            lhs = x_ref[pl.ds(4 * c + kh, 4)].reshape(4 * S, 96)
            d = jnp.dot(lhs, t1_ref[kh], preferred_element_type=jnp.float32,
                        precision=_PREC)
            acc = d if acc is None else acc + d
        zz = acc.reshape(4, S, 112)
        for q in range(2):
            u = jnp.maximum(zz[2 * q], zz[2 * q + 1])
            v = jnp.maximum(u[:, :56], u[:, 56:])
            p1_ref[2 * c + q] = jnp.maximum(v + b1_ref[...], 0.0)

    for j in range(5):
        acc = None
        for kh in range(5):
            lhs = p1_ref[pl.ds(2 * j + kh, 2)].reshape(2 * S, 56)
            d = jnp.dot(lhs, t2_ref[kh], preferred_element_type=jnp.float32,
                        precision=_PREC)
            acc = d if acc is None else acc + d
        zz = acc.reshape(2, S, 120)
        u = jnp.maximum(zz[0], zz[1])
        v = jnp.maximum(u[:, :60], u[:, 60:])
        p2_ref[j] = jnp.maximum(v + b2_ref[...], 0.0)

    y = bf1_ref[...]
    for h in range(5):
        y = y + jnp.dot(p2_ref[h], wf1_ref[h], preferred_element_type=jnp.float32,
                        precision=_PREC)
    y = jnp.maximum(y, 0.0)

    f = jnp.dot(y, wf2_ref[...], preferred_element_type=jnp.float32,
                precision=_PREC) + bf2_ref[...]

    z = jnp.dot(f, hwa_ref[...], preferred_element_type=jnp.float32,
                precision=_PREC)
    xn = jnp.maximum(jnp.sqrt(jnp.sum(f * f, axis=-1, keepdims=True)), 1e-8)
    hcos = z[:, :N_CLASSES] / (xn * wn_ref[...])
    gl = z[:, N_CLASSES:N_CLASSES + 1] + gb_ref[...]
    o_ref[...] = jnp.concatenate([hcos, gl], axis=1)


def _head_kernel(a_ref, o_ref):
    a = a_ref[...]
    gl = a[:, N_CLASSES:N_CLASSES + 1]
    h = a[:, :N_CLASSES]
    mu = jnp.mean(gl, axis=0, keepdims=True)
    var = jnp.mean((gl - mu) ** 2, axis=0, keepdims=True)
    g = jax.nn.sigmoid((gl - mu) * jax.lax.rsqrt(var + 1e-5))
    out = g / h
    out = out - jnp.max(out, axis=-1, keepdims=True)
    e = jnp.exp(out)
    o_ref[...] = e / jnp.sum(e, axis=-1, keepdims=True)


@jax.jit
def _forward(x, w1, b1, w2, b2, wf1, bf1, wf2, bf2, hwa, gb, wn):
    B = x.shape[0]
    S = _S
    xr = jnp.transpose(x, (2, 0, 1, 3)).reshape(32, B, 96)
    t1 = jnp.transpose(w1, (0, 2, 1, 3)).reshape(5, 32, 3, 112)
    t1 = jnp.transpose(t1, (0, 2, 1, 3)).reshape(5, 96, 112)
    t2 = jnp.transpose(w2, (0, 2, 1, 3)).reshape(5, 56, 120)

    part = pl.pallas_call(
        _feat_kernel,
        out_shape=jax.ShapeDtypeStruct((B, N_CLASSES + 1), jnp.float32),
        grid=(B // S,),
        in_specs=[
            pl.BlockSpec((32, S, 96), lambda i: (0, i, 0)),
            pl.BlockSpec((5, 96, 112), lambda i: (0, 0, 0)),
            pl.BlockSpec((1, 56), lambda i: (0, 0)),
            pl.BlockSpec((5, 56, 120), lambda i: (0, 0, 0)),
            pl.BlockSpec((1, 60), lambda i: (0, 0)),
            pl.BlockSpec((5, 60, 120), lambda i: (0, 0, 0)),
            pl.BlockSpec((1, 120), lambda i: (0, 0)),
            pl.BlockSpec((120, 64), lambda i: (0, 0)),
            pl.BlockSpec((1, 64), lambda i: (0, 0)),
            pl.BlockSpec((64, N_CLASSES + 1), lambda i: (0, 0)),
            pl.BlockSpec((1, 1), lambda i: (0, 0)),
            pl.BlockSpec((1, N_CLASSES), lambda i: (0, 0)),
        ],
        out_specs=pl.BlockSpec((S, N_CLASSES + 1), lambda i: (i, 0)),
        scratch_shapes=[pltpu.VMEM((14, S, 56), jnp.float32),
                        pltpu.VMEM((5, S, 60), jnp.float32)],
        compiler_params=pltpu.CompilerParams(
            dimension_semantics=("parallel",)),
    )(xr, t1, b1, t2, b2, wf1, bf1, wf2, bf2, hwa, gb, wn)

    pred = pl.pallas_call(
        _head_kernel,
        out_shape=jax.ShapeDtypeStruct((B, N_CLASSES), jnp.float32),
        grid=(1,),
        in_specs=[pl.BlockSpec((B, N_CLASSES + 1), lambda i: (0, 0))],
        out_specs=pl.BlockSpec((B, N_CLASSES), lambda i: (0, 0)),
        compiler_params=pltpu.CompilerParams(
            dimension_semantics=("arbitrary",)),
    )(part)
    return pred


def kernel(x, w1, b1, w2, b2, wf1, bf1, wf2, bf2, hwa, gb, wn):
    return _forward(x, w1, b1, w2, b2, wf1, bf1, wf2, bf2, hwa, gb, wn)


# speedup vs baseline: 1.3401x; 1.1764x over previous
import jax
import jax.numpy as jnp
from jax.experimental import pallas as pl
from jax.experimental.pallas import tpu as pltpu

N_CLASSES = 10
_S = 128
_PREC = jax.lax.Precision.HIGHEST


def _feat_kernel(x_ref, t1_ref, b1_ref, t2_ref, b2_ref,
                 wf1_ref, bf1_ref, wf2_ref, bf2_ref, hwa_ref, gb_ref, wn_ref,
                 o_ref, p1_ref, p2_ref):
    S = x_ref.shape[1]

    for c in range(7):
        acc = None
        for kh in range(5):
            lhs = x_ref[pl.ds(4 * c + kh, 4)].reshape(4 * S, 96)
            d = jnp.dot(lhs, t1_ref[kh], preferred_element_type=jnp.float32,
                        precision=_PREC)
            acc = d if acc is None else acc + d
        zz = acc.reshape(4, S, 112)
        for q in range(2):
            u = jnp.maximum(zz[2 * q], zz[2 * q + 1])
            v = jnp.maximum(u[:, :56], u[:, 56:])
            p1_ref[2 * c + q] = jnp.maximum(v + b1_ref[...], 0.0)

    for j in range(5):
        acc = None
        for kh in range(5):
            lhs = p1_ref[pl.ds(2 * j + kh, 2)].reshape(2 * S, 56)
            d = jnp.dot(lhs, t2_ref[kh], preferred_element_type=jnp.float32,
                        precision=_PREC)
            acc = d if acc is None else acc + d
        zz = acc.reshape(2, S, 120)
        u = jnp.maximum(zz[0], zz[1])
        v = jnp.maximum(u[:, :60], u[:, 60:])
        p2_ref[j] = jnp.maximum(v + b2_ref[...], 0.0)

    y = bf1_ref[...]
    for h in range(5):
        y = y + jnp.dot(p2_ref[h], wf1_ref[h], preferred_element_type=jnp.float32,
                        precision=_PREC)
    y = jnp.maximum(y, 0.0)

    f = jnp.dot(y, wf2_ref[...], preferred_element_type=jnp.float32,
                precision=_PREC) + bf2_ref[...]

    z = jnp.dot(f, hwa_ref[...], preferred_element_type=jnp.float32,
                precision=_PREC)
    xn = jnp.maximum(jnp.sqrt(jnp.sum(f * f, axis=-1, keepdims=True)), 1e-8)
    hcos = z[:, :N_CLASSES] / (xn * wn_ref[...])
    gl = z[:, N_CLASSES:N_CLASSES + 1] + gb_ref[...]
    o_ref[...] = jnp.concatenate([hcos, gl], axis=1)


def _head_kernel(a_ref, o_ref):
    a = a_ref[...]
    gl = a[:, N_CLASSES:N_CLASSES + 1]
    h = a[:, :N_CLASSES]
    mu = jnp.mean(gl, axis=0, keepdims=True)
    var = jnp.mean((gl - mu) ** 2, axis=0, keepdims=True)
    g = jax.nn.sigmoid((gl - mu) * jax.lax.rsqrt(var + 1e-5))
    out = g / h
    out = out - jnp.max(out, axis=-1, keepdims=True)
    e = jnp.exp(out)
    o_ref[...] = e / jnp.sum(e, axis=-1, keepdims=True)


@jax.jit
def _forward(x, w1, b1, w2, b2, wf1, bf1, wf2, bf2, hwa, gb, wn):
    B = x.shape[0]
    S = _S
    xr = jnp.transpose(x, (2, 0, 1, 3)).reshape(32, B, 96)
    t1 = jnp.transpose(w1, (0, 2, 1, 3)).reshape(5, 32, 3, 112)
    t1 = jnp.transpose(t1, (0, 2, 1, 3)).reshape(5, 96, 112)
    t2 = jnp.transpose(w2, (0, 2, 1, 3)).reshape(5, 56, 120)

    part = pl.pallas_call(
        _feat_kernel,
        out_shape=jax.ShapeDtypeStruct((B, N_CLASSES + 1), jnp.float32),
        grid=(B // S,),
        in_specs=[
            pl.BlockSpec((32, S, 96), lambda i: (0, i, 0)),
            pl.BlockSpec((5, 96, 112), lambda i: (0, 0, 0)),
            pl.BlockSpec((1, 56), lambda i: (0, 0)),
            pl.BlockSpec((5, 56, 120), lambda i: (0, 0, 0)),
            pl.BlockSpec((1, 60), lambda i: (0, 0)),
            pl.BlockSpec((5, 60, 120), lambda i: (0, 0, 0)),
            pl.BlockSpec((1, 120), lambda i: (0, 0)),
            pl.BlockSpec((120, 64), lambda i: (0, 0)),
            pl.BlockSpec((1, 64), lambda i: (0, 0)),
            pl.BlockSpec((64, N_CLASSES + 1), lambda i: (0, 0)),
            pl.BlockSpec((1, 1), lambda i: (0, 0)),
            pl.BlockSpec((1, N_CLASSES), lambda i: (0, 0)),
        ],
        out_specs=pl.BlockSpec((S, N_CLASSES + 1), lambda i: (i, 0)),
        scratch_shapes=[pltpu.VMEM((14, S, 56), jnp.float32),
                        pltpu.VMEM((5, S, 60), jnp.float32)],
        compiler_params=pltpu.CompilerParams(
            dimension_semantics=("parallel",)),
    )(xr, t1, b1, t2, b2, wf1, bf1, wf2, bf2, hwa, gb, wn)

    pred = pl.pallas_call(
        _head_kernel,
        out_shape=jax.ShapeDtypeStruct((B, N_CLASSES), jnp.float32),
        grid=(1,),
        in_specs=[pl.BlockSpec((B, N_CLASSES + 1), lambda i: (0, 0))],
        out_specs=pl.BlockSpec((B, N_CLASSES), lambda i: (0, 0)),
        compiler_params=pltpu.CompilerParams(
            dimension_semantics=("arbitrary",)),
    )(part)
    return pred


def kernel(x, w1, b1, w2, b2, wf1, bf1, wf2, bf2, hwa, gb, wn):
    return _forward(x, w1, b1, w2, b2, wf1, bf1, wf2, bf2, hwa, gb, wn)
